# baseline (speedup 1.0000x reference)
import functools

import jax
import jax.numpy as jnp
from jax import lax
from jax.experimental import pallas as pl
from jax.experimental.pallas import tpu as pltpu
from jax.experimental.pallas import tpu_sc as plsc

N = 10000
E = 320000
D = 128

NC = 2
NS = 16
NW = NC * NS
EPW = E // NW
NSETS = 5
SCH = 40
SNCH = EPW // SCH
NSLAB = 1
SLAB = E // NSLAB
SL_EPW = SLAB // NW
GCH = 80
GNCH = SL_EPW // GCH
SITER = SNCH // NSETS
GITER = GNCH // NSETS
RPS = 624
REM = N - RPS * NS

_sc_mesh = plsc.VectorSubcoreMesh(
    core_axis_name="c", subcore_axis_name="s", num_cores=NC, num_subcores=NS)


def _worker_id():
    return lax.axis_index("s") * NC + lax.axis_index("c")


def _drain(n, dummy_hbm, buf_ref, sem):
    for _ in range(n):
        pltpu.make_async_copy(dummy_hbm, buf_ref, sem).wait()


def _make_scatter():
    def body(edges_hbm, dst_hbm, zeros_hbm, out_hbm, idx_v, rows, table,
             *sems):
        cid = lax.axis_index("c")
        sid = lax.axis_index("s")
        wid = _worker_id()
        sem_l, sem_a = sems[:NSETS], sems[NSETS:]

        pltpu.sync_copy(zeros_hbm.at[pl.ds(sid * RPS, RPS)],
                        table.at[pl.ds(sid * RPS, RPS)])

        @pl.when(sid == NS - 1)
        def _():
            pltpu.sync_copy(zeros_hbm.at[pl.ds(RPS * NS, REM)],
                            table.at[pl.ds(RPS * NS, REM)])

        plsc.subcore_barrier()

        base = wid * EPW

        def fire_load(c, p):
            pltpu.async_copy(edges_hbm.at[pl.ds(base + c * SCH, SCH)],
                             rows.at[p], sem_l[p])
            pltpu.async_copy(dst_hbm.at[wid, c], idx_v[p], sem_l[p])

        fire_load(0, 0)

        def grp(g, carry):
            for p in range(NSETS):
                c = g * NSETS + p
                p1 = (p + 1) % NSETS
                _drain(1, edges_hbm.at[pl.ds(0, SCH)], rows.at[p], sem_l[p])
                _drain(1, dst_hbm.at[wid, 0], idx_v[p], sem_l[p])
                pltpu.async_copy(rows.at[p], table.at[idx_v[p]],
                                 sem_a[p], add=True)

                if p == NSETS - 1:
                    @pl.when(g < SITER - 1)
                    def _():
                        _drain(1, edges_hbm.at[pl.ds(0, SCH)], rows.at[p1],
                               sem_a[p1])
                        fire_load(c + 1, p1)
                else:
                    @pl.when(g > 0)
                    def _():
                        _drain(1, edges_hbm.at[pl.ds(0, SCH)], rows.at[p1],
                               sem_a[p1])
                    fire_load(c + 1, p1)
            return carry

        lax.fori_loop(0, SITER, grp, 0)
        for p in range(NSETS):
            _drain(1, edges_hbm.at[pl.ds(0, SCH)], rows.at[p], sem_a[p])

        plsc.subcore_barrier()
        pltpu.sync_copy(table.at[pl.ds(sid * RPS, RPS)],
                        out_hbm.at[pl.ds(cid * N + sid * RPS, RPS)])

        @pl.when(sid == NS - 1)
        def _():
            pltpu.sync_copy(table.at[pl.ds(RPS * NS, REM)],
                            out_hbm.at[pl.ds(cid * N + RPS * NS, REM)])

    return functools.partial(
        pl.kernel,
        out_type=jax.ShapeDtypeStruct((NC * N, D), jnp.float32),
        mesh=_sc_mesh,
        scratch_types=[
            [pltpu.VMEM((SCH,), jnp.int32) for _ in range(NSETS)],
            pltpu.VMEM((NSETS, SCH, D), jnp.float32),
            pltpu.VMEM_SHARED((N, D), jnp.float32),
        ] + [pltpu.SemaphoreType.DMA] * (2 * NSETS),
    )(body)


_scatter_add = _make_scatter()


def _make_gather():
    def body(a_hbm, b_hbm, gidx_hbm, out_s, idx2, rows, *sems):
        wid = _worker_id()
        sem_g, sem_i, sem_w = sems[:NSETS], sems[NSETS:2 * NSETS], \
            sems[2 * NSETS:]
        base = wid * SL_EPW

        def fire_idx(c, p):
            pltpu.async_copy(gidx_hbm.at[wid, c], idx2[p], sem_i[p])

        def finish_chunk(c, p):
            _drain(2, a_hbm.at[pl.ds(0, GCH)], rows.at[p, 0], sem_g[p])

            def radd(r, carry):
                for j in range(D // 16):
                    sl = pl.ds(j * 16, 16)
                    rows[p, 0, r, sl] = rows[p, 0, r, sl] + rows[p, 1, r, sl]
                return carry

            lax.fori_loop(0, GCH, radd, 0)
            off = base + c * GCH
            pltpu.async_copy(rows.at[p, 0], out_s.at[pl.ds(off, GCH)],
                             sem_w[p])

        fire_idx(0, 0)

        def grp(g, carry):
            for p in range(NSETS):
                c = g * NSETS + p
                p1 = (p + 1) % NSETS
                pm = (p - 1) % NSETS

                @pl.when(g > 0)
                def _():
                    _drain(1, a_hbm.at[pl.ds(0, GCH)], rows.at[p, 0],
                           sem_w[p])

                _drain(1, gidx_hbm.at[wid, 0], idx2[p], sem_i[p])
                pltpu.async_copy(a_hbm.at[idx2[p].at[0]], rows.at[p, 0],
                                 sem_g[p])
                pltpu.async_copy(b_hbm.at[idx2[p].at[1]], rows.at[p, 1],
                                 sem_g[p])

                if p == NSETS - 1:
                    @pl.when(g < GITER - 1)
                    def _():
                        fire_idx(c + 1, p1)
                else:
                    fire_idx(c + 1, p1)

                pm2 = (p - 2) % NSETS
                if p <= 1:
                    @pl.when(g > 0)
                    def _():
                        finish_chunk(c - 2, pm2)
                else:
                    finish_chunk(c - 2, pm2)
            return carry

        lax.fori_loop(0, GITER, grp, 0)
        finish_chunk(GNCH - 2, (GNCH - 2) % NSETS)
        finish_chunk(GNCH - 1, (GNCH - 1) % NSETS)
        for p in range(NSETS):
            _drain(1, a_hbm.at[pl.ds(0, GCH)], rows.at[p, 0], sem_w[p])

    return functools.partial(
        pl.kernel,
        out_type=jax.ShapeDtypeStruct((SLAB, D), jnp.float32),
        mesh=_sc_mesh,
        scratch_types=[
            [pltpu.VMEM((2, GCH), jnp.int32) for _ in range(NSETS)],
            pltpu.VMEM((NSETS, 2, GCH, D), jnp.float32),
        ] + [pltpu.SemaphoreType.DMA] * (3 * NSETS),
    )(body)


_dual_gather = _make_gather()


NODE_BLK = 5000
EDGE_BLK = 8000


def _node_body(nodes_ref, m0_ref, m1_ref, nW1a_ref, nW1b_ref, nb1_ref,
               ng1_ref, ngb1_ref, nW2_ref, nb2_ref, eW1s_ref, eW1d_ref,
               out_ref, a_ref, b_ref):
    x = nodes_ref[...]
    m = m0_ref[...] + m1_ref[...]
    h = (jnp.dot(x, nW1a_ref[...], preferred_element_type=jnp.float32)
         + jnp.dot(m, nW1b_ref[...], preferred_element_type=jnp.float32)
         + nb1_ref[...])
    h = jnp.maximum(h, 0.0)
    mu = jnp.mean(h, axis=-1, keepdims=True)
    var = jnp.mean(h * h, axis=-1, keepdims=True) - mu * mu
    h = (h - mu) * (lax.rsqrt(var + 1e-5) * ng1_ref[...]) + ngb1_ref[...]
    out = jnp.maximum(
        jnp.dot(h, nW2_ref[...], preferred_element_type=jnp.float32)
        + nb2_ref[...], 0.0) + x
    out_ref[...] = out
    a_ref[...] = jnp.dot(out, eW1s_ref[...], preferred_element_type=jnp.float32)
    b_ref[...] = jnp.dot(out, eW1d_ref[...], preferred_element_type=jnp.float32)


def _edge_body(s_ref, e_ref, eb1_ref, eg1_ref, egb1_ref,
               eW1e_ref, eW2_ref, eb2_ref, out_ref):
    e = e_ref[...]
    h = (s_ref[...]
         + jnp.dot(e, eW1e_ref[...], preferred_element_type=jnp.float32)
         + eb1_ref[...])
    h = jnp.maximum(h, 0.0)
    mu = jnp.mean(h, axis=-1, keepdims=True)
    var = jnp.mean(h * h, axis=-1, keepdims=True) - mu * mu
    h = (h - mu) * (lax.rsqrt(var + 1e-5) * eg1_ref[...]) + egb1_ref[...]
    out_ref[...] = jnp.tanh(
        jnp.dot(h, eW2_ref[...], preferred_element_type=jnp.float32)
        + eb2_ref[...]) + e


def _row_spec(blk):
    return pl.BlockSpec((blk, D), lambda i: (i, 0))


def _full_spec(shape):
    return pl.BlockSpec(shape, lambda i: tuple(0 for _ in shape))


def _node_mlp(nodes, m0, m1, nW1a, nW1b, nb1, ng1, ngb1, nW2, nb2, eW1s, eW1d):
    grid = (N // NODE_BLK,)
    return pl.pallas_call(
        _node_body,
        grid=grid,
        in_specs=[
            _row_spec(NODE_BLK), _row_spec(NODE_BLK), _row_spec(NODE_BLK),
            _full_spec((D, D)), _full_spec((D, D)), _full_spec((1, D)),
            _full_spec((1, D)), _full_spec((1, D)), _full_spec((D, D)),
            _full_spec((1, D)), _full_spec((D, D)), _full_spec((D, D)),
        ],
        out_specs=[_row_spec(NODE_BLK), _row_spec(NODE_BLK),
                   _row_spec(NODE_BLK)],
        out_shape=[jax.ShapeDtypeStruct((N, D), jnp.float32)] * 3,
    )(nodes, m0, m1, nW1a, nW1b, nb1, ng1, ngb1, nW2, nb2, eW1s, eW1d)


def _edge_mlp(s_g, edges, eb1, eg1, egb1, eW1e, eW2, eb2):
    grid = (E // EDGE_BLK,)
    return pl.pallas_call(
        _edge_body,
        grid=grid,
        in_specs=[
            _row_spec(EDGE_BLK), _row_spec(EDGE_BLK),
            _full_spec((1, D)), _full_spec((1, D)), _full_spec((1, D)),
            _full_spec((D, D)), _full_spec((D, D)), _full_spec((1, D)),
        ],
        out_specs=_row_spec(EDGE_BLK),
        out_shape=jax.ShapeDtypeStruct((E, D), jnp.float32),
    )(s_g, edges, eb1, eg1, egb1, eW1e, eW2, eb2)


def kernel(nodes, edges, graph, nW1, nb1, ng1, ngb1, nW2, nb2,
           eW1, eb1, eg1, egb1, eW2, eb2):
    src32 = graph[0].astype(jnp.int32)
    dst32 = graph[1].astype(jnp.int32)
    dst_s = dst32.reshape(NW, SNCH, SCH)
    gidx = jnp.stack([src32.reshape(NW, GNCH, GCH),
                      dst32.reshape(NW, GNCH, GCH)], axis=2)
    zeros = jnp.zeros((N, D), jnp.float32)

    parts = _scatter_add(edges, dst_s, zeros)
    m0, m1 = parts[:N], parts[N:]

    nW1a, nW1b = nW1[:D], nW1[D:]
    eW1s, eW1d, eW1e = eW1[:D], eW1[D:2 * D], eW1[2 * D:]
    r = lambda v: v.reshape(1, D)

    nodes_new, a_tab, b_tab = _node_mlp(
        nodes, m0, m1, nW1a, nW1b, r(nb1), r(ng1), r(ngb1), nW2, r(nb2),
        eW1s, eW1d)

    s_g = _dual_gather(a_tab, b_tab, gidx)
    edges_new = _edge_mlp(s_g, edges, r(eb1), r(eg1), r(egb1),
                          eW1e, eW2, r(eb2))
    return (nodes_new, edges_new)


# speedup vs baseline: 1.0057x; 1.0057x over previous
import functools

import jax
import jax.numpy as jnp
from jax import lax
from jax.experimental import pallas as pl
from jax.experimental.pallas import tpu as pltpu
from jax.experimental.pallas import tpu_sc as plsc

N = 10000
E = 320000
D = 128

NC = 2
NS = 16
NW = NC * NS
EPW = E // NW
NSETS = 5
SCH = 40
SNCH = EPW // SCH
NSLAB = 1
SLAB = E // NSLAB
SL_EPW = SLAB // NW
GCH = 80
GNCH = SL_EPW // GCH
SITER = SNCH // NSETS
GITER = GNCH // NSETS
RPS = 624
REM = N - RPS * NS

_sc_mesh = plsc.VectorSubcoreMesh(
    core_axis_name="c", subcore_axis_name="s", num_cores=NC, num_subcores=NS)


def _worker_id():
    return lax.axis_index("s") * NC + lax.axis_index("c")


def _drain(n, dummy_hbm, buf_ref, sem):
    for _ in range(n):
        pltpu.make_async_copy(dummy_hbm, buf_ref, sem).wait()


def _make_scatter():
    def body(edges_hbm, dst_hbm, zeros_hbm, out_hbm, idx_v, rows, table,
             *sems):
        cid = lax.axis_index("c")
        sid = lax.axis_index("s")
        wid = _worker_id()
        sem_l, sem_a = sems[:NSETS], sems[NSETS:]

        pltpu.sync_copy(zeros_hbm.at[pl.ds(sid * RPS, RPS)],
                        table.at[pl.ds(sid * RPS, RPS)])

        @pl.when(sid == NS - 1)
        def _():
            pltpu.sync_copy(zeros_hbm.at[pl.ds(RPS * NS, REM)],
                            table.at[pl.ds(RPS * NS, REM)])

        plsc.subcore_barrier()

        base = wid * EPW

        def fire_load(c, p):
            pltpu.async_copy(edges_hbm.at[pl.ds(base + c * SCH, SCH)],
                             rows.at[p], sem_l[p])
            pltpu.async_copy(dst_hbm.at[wid, c], idx_v[p], sem_l[p])

        fire_load(0, 0)

        def grp(g, carry):
            for p in range(NSETS):
                c = g * NSETS + p
                p1 = (p + 1) % NSETS
                _drain(1, edges_hbm.at[pl.ds(0, SCH)], rows.at[p], sem_l[p])
                _drain(1, dst_hbm.at[wid, 0], idx_v[p], sem_l[p])
                pltpu.async_copy(rows.at[p], table.at[idx_v[p]],
                                 sem_a[p], add=True)

                if p == NSETS - 1:
                    @pl.when(g < SITER - 1)
                    def _():
                        _drain(1, edges_hbm.at[pl.ds(0, SCH)], rows.at[p1],
                               sem_a[p1])
                        fire_load(c + 1, p1)
                else:
                    @pl.when(g > 0)
                    def _():
                        _drain(1, edges_hbm.at[pl.ds(0, SCH)], rows.at[p1],
                               sem_a[p1])
                    fire_load(c + 1, p1)
            return carry

        lax.fori_loop(0, SITER, grp, 0)
        for p in range(NSETS):
            _drain(1, edges_hbm.at[pl.ds(0, SCH)], rows.at[p], sem_a[p])

        plsc.subcore_barrier()
        pltpu.sync_copy(table.at[pl.ds(sid * RPS, RPS)],
                        out_hbm.at[pl.ds(cid * N + sid * RPS, RPS)])

        @pl.when(sid == NS - 1)
        def _():
            pltpu.sync_copy(table.at[pl.ds(RPS * NS, REM)],
                            out_hbm.at[pl.ds(cid * N + RPS * NS, REM)])

    return functools.partial(
        pl.kernel,
        out_type=jax.ShapeDtypeStruct((NC * N, D), jnp.float32),
        mesh=_sc_mesh,
        scratch_types=[
            [pltpu.VMEM((SCH,), jnp.int32) for _ in range(NSETS)],
            pltpu.VMEM((NSETS, SCH, D), jnp.float32),
            pltpu.VMEM_SHARED((N, D), jnp.float32),
        ] + [pltpu.SemaphoreType.DMA] * (2 * NSETS),
    )(body)


_scatter_add = _make_scatter()


def _make_gather():
    def body(a_hbm, b_hbm, gidx_hbm, out_s, idx2, rows, *sems):
        wid = _worker_id()
        sem_g, sem_i, sem_w = sems[:NSETS], sems[NSETS:2 * NSETS], \
            sems[2 * NSETS:]
        base = wid * SL_EPW

        def fire_idx(c, p):
            pltpu.async_copy(gidx_hbm.at[wid, c], idx2[p], sem_i[p])

        def finish_chunk(c, p):
            _drain(2, a_hbm.at[pl.ds(0, GCH)], rows.at[p, 0], sem_g[p])

            def radd(r, carry):
                for j in range(D // 16):
                    sl = pl.ds(j * 16, 16)
                    rows[p, 0, r, sl] = rows[p, 0, r, sl] + rows[p, 1, r, sl]
                return carry

            lax.fori_loop(0, GCH, radd, 0)
            off = base + c * GCH
            pltpu.async_copy(rows.at[p, 0], out_s.at[pl.ds(off, GCH)],
                             sem_w[p])

        fire_idx(0, 0)

        def grp(g, carry):
            for p in range(NSETS):
                c = g * NSETS + p
                p1 = (p + 1) % NSETS
                pm = (p - 1) % NSETS

                @pl.when(g > 0)
                def _():
                    _drain(1, a_hbm.at[pl.ds(0, GCH)], rows.at[p, 0],
                           sem_w[p])

                _drain(1, gidx_hbm.at[wid, 0], idx2[p], sem_i[p])
                pltpu.async_copy(a_hbm.at[idx2[p].at[0]], rows.at[p, 0],
                                 sem_g[p])
                pltpu.async_copy(b_hbm.at[idx2[p].at[1]], rows.at[p, 1],
                                 sem_g[p])

                if p == NSETS - 1:
                    @pl.when(g < GITER - 1)
                    def _():
                        fire_idx(c + 1, p1)
                else:
                    fire_idx(c + 1, p1)

                pm2 = (p - 2) % NSETS
                if p <= 1:
                    @pl.when(g > 0)
                    def _():
                        finish_chunk(c - 2, pm2)
                else:
                    finish_chunk(c - 2, pm2)
            return carry

        lax.fori_loop(0, GITER, grp, 0)
        finish_chunk(GNCH - 2, (GNCH - 2) % NSETS)
        finish_chunk(GNCH - 1, (GNCH - 1) % NSETS)
        for p in range(NSETS):
            _drain(1, a_hbm.at[pl.ds(0, GCH)], rows.at[p, 0], sem_w[p])

    return functools.partial(
        pl.kernel,
        out_type=jax.ShapeDtypeStruct((SLAB, D), jnp.float32),
        mesh=_sc_mesh,
        scratch_types=[
            [pltpu.VMEM((2, GCH), jnp.int32) for _ in range(NSETS)],
            pltpu.VMEM((NSETS, 2, GCH, D), jnp.float32),
        ] + [pltpu.SemaphoreType.DMA] * (3 * NSETS),
    )(body)


_dual_gather = _make_gather()


NODE_BLK = 5000
EDGE_BLK = 8000


def _node_body(nodes_ref, m0_ref, m1_ref, nW1a_ref, nW1b_ref, nb1_ref,
               ng1_ref, ngb1_ref, nW2_ref, nb2_ref, eW1s_ref, eW1d_ref,
               out_ref, a_ref, b_ref):
    x = nodes_ref[...]
    m = m0_ref[...] + m1_ref[...]
    h = (jnp.dot(x, nW1a_ref[...], preferred_element_type=jnp.float32)
         + jnp.dot(m, nW1b_ref[...], preferred_element_type=jnp.float32)
         + nb1_ref[...])
    h = jnp.maximum(h, 0.0)
    mu = jnp.mean(h, axis=-1, keepdims=True)
    var = jnp.mean((h - mu) ** 2, axis=-1, keepdims=True)
    h = (h - mu) * lax.rsqrt(var + 1e-5) * ng1_ref[...] + ngb1_ref[...]
    out = jnp.maximum(
        jnp.dot(h, nW2_ref[...], preferred_element_type=jnp.float32)
        + nb2_ref[...], 0.0) + x
    out_ref[...] = out
    a_ref[...] = jnp.dot(out, eW1s_ref[...], preferred_element_type=jnp.float32)
    b_ref[...] = jnp.dot(out, eW1d_ref[...], preferred_element_type=jnp.float32)


def _edge_body(s_ref, e_ref, eb1_ref, eg1_ref, egb1_ref,
               eW1e_ref, eW2_ref, eb2_ref, out_ref):
    e = e_ref[...]
    h = (s_ref[...]
         + jnp.dot(e, eW1e_ref[...], preferred_element_type=jnp.float32)
         + eb1_ref[...])
    h = jnp.maximum(h, 0.0)
    mu = jnp.mean(h, axis=-1, keepdims=True)
    var = jnp.mean((h - mu) ** 2, axis=-1, keepdims=True)
    h = (h - mu) * lax.rsqrt(var + 1e-5) * eg1_ref[...] + egb1_ref[...]
    out_ref[...] = jnp.tanh(
        jnp.dot(h, eW2_ref[...], preferred_element_type=jnp.float32)
        + eb2_ref[...]) + e


def _row_spec(blk):
    return pl.BlockSpec((blk, D), lambda i: (i, 0))


def _full_spec(shape):
    return pl.BlockSpec(shape, lambda i: tuple(0 for _ in shape))


def _node_mlp(nodes, m0, m1, nW1a, nW1b, nb1, ng1, ngb1, nW2, nb2, eW1s, eW1d):
    grid = (N // NODE_BLK,)
    return pl.pallas_call(
        _node_body,
        grid=grid,
        in_specs=[
            _row_spec(NODE_BLK), _row_spec(NODE_BLK), _row_spec(NODE_BLK),
            _full_spec((D, D)), _full_spec((D, D)), _full_spec((1, D)),
            _full_spec((1, D)), _full_spec((1, D)), _full_spec((D, D)),
            _full_spec((1, D)), _full_spec((D, D)), _full_spec((D, D)),
        ],
        out_specs=[_row_spec(NODE_BLK), _row_spec(NODE_BLK),
                   _row_spec(NODE_BLK)],
        out_shape=[jax.ShapeDtypeStruct((N, D), jnp.float32)] * 3,
    )(nodes, m0, m1, nW1a, nW1b, nb1, ng1, ngb1, nW2, nb2, eW1s, eW1d)


def _edge_mlp(s_g, edges, eb1, eg1, egb1, eW1e, eW2, eb2):
    grid = (E // EDGE_BLK,)
    return pl.pallas_call(
        _edge_body,
        grid=grid,
        in_specs=[
            _row_spec(EDGE_BLK), _row_spec(EDGE_BLK),
            _full_spec((1, D)), _full_spec((1, D)), _full_spec((1, D)),
            _full_spec((D, D)), _full_spec((D, D)), _full_spec((1, D)),
        ],
        out_specs=_row_spec(EDGE_BLK),
        out_shape=jax.ShapeDtypeStruct((E, D), jnp.float32),
    )(s_g, edges, eb1, eg1, egb1, eW1e, eW2, eb2)


def kernel(nodes, edges, graph, nW1, nb1, ng1, ngb1, nW2, nb2,
           eW1, eb1, eg1, egb1, eW2, eb2):
    src32 = graph[0].astype(jnp.int32)
    dst32 = graph[1].astype(jnp.int32)
    dst_s = dst32.reshape(NW, SNCH, SCH)
    gidx = jnp.stack([src32.reshape(NW, GNCH, GCH),
                      dst32.reshape(NW, GNCH, GCH)], axis=2)
    zeros = jnp.zeros((N, D), jnp.float32)

    parts = _scatter_add(edges, dst_s, zeros)
    m0, m1 = parts[:N], parts[N:]

    nW1a, nW1b = nW1[:D], nW1[D:]
    eW1s, eW1d, eW1e = eW1[:D], eW1[D:2 * D], eW1[2 * D:]
    r = lambda v: v.reshape(1, D)

    nodes_new, a_tab, b_tab = _node_mlp(
        nodes, m0, m1, nW1a, nW1b, r(nb1), r(ng1), r(ngb1), nW2, r(nb2),
        eW1s, eW1d)

    s_g = _dual_gather(a_tab, b_tab, gidx)
    edges_new = _edge_mlp(s_g, edges, r(eb1), r(eg1), r(egb1),
                          eW1e, eW2, r(eb2))
    return (nodes_new, edges_new)
